# baseline (speedup 1.0000x reference)
import functools

import jax
import jax.numpy as jnp
from jax import lax
from jax.experimental import pallas as pl
from jax.experimental.pallas import tpu as pltpu
from jax.experimental.pallas import tpu_sc as plsc

D = 32
SUB = 128
K = 8
CHUNK = K * SUB
NC = 2
NS = 16
NW = NC * NS


def _sc_detile(table_t, tail):
    d, v = table_t.shape
    n_blk = v // SUB
    per_w = -(-n_blk // NW)
    tail_elems = (v - n_blk * SUB) * d

    mesh = plsc.VectorSubcoreMesh(core_axis_name="c", subcore_axis_name="s")

    @functools.partial(
        pl.kernel,
        mesh=mesh,
        out_type=jax.ShapeDtypeStruct((v * d,), jnp.float32),
        scratch_types=[
            pltpu.VMEM((d // 8, 8, SUB), jnp.float32),
            pltpu.VMEM((SUB * d,), jnp.float32),
            pltpu.VMEM((tail_elems,), jnp.float32),
        ],
        compiler_params=pltpu.CompilerParams(
            use_tc_tiling_on_sc=True, needs_layout_passes=False
        ),
    )
    def k(tt_hbm, tail_hbm, out_hbm, src, dst, tbuf):
        wid = lax.axis_index("s") * NC + lax.axis_index("c")
        i16 = lax.iota(jnp.int32, 16)

        def blk(i, carry):
            c = wid * per_w + i

            @pl.when(c < n_blk)
            def _():
                for tr in range(d // 8):
                    pltpu.sync_copy(
                        tt_hbm.at[pl.ds(tr * 8, 8), pl.ds(c * SUB, SUB)],
                        src.at[tr],
                    )

                def rows(c0g, carry2):
                    base = (c0g * 16 + i16) * d
                    for f in range(d):
                        vals = src[f // 8, f % 8, pl.ds(c0g * 16, 16)]
                        plsc.store_scatter(dst, [base + f], vals)
                    return carry2

                lax.fori_loop(0, SUB // 16, rows, 0)
                pltpu.sync_copy(dst, out_hbm.at[pl.ds(c * SUB * d, SUB * d)])

            return carry

        lax.fori_loop(0, per_w, blk, 0)

        @pl.when(wid == NW - 1)
        def _():
            pltpu.sync_copy(tail_hbm, tbuf)
            pltpu.sync_copy(
                tbuf, out_hbm.at[pl.ds(n_blk * SUB * d, tail_elems)]
            )

    return k(table_t, tail)


def _sc_gather(table, idx2d):
    n_rows = idx2d.shape[0]
    rows_per_w = n_rows // NW
    n_chunks = rows_per_w // K
    n_pairs = (n_chunks - 1) // 2
    b_flat = n_rows * SUB

    mesh = plsc.VectorSubcoreMesh(core_axis_name="c", subcore_axis_name="s")

    @functools.partial(
        pl.kernel,
        mesh=mesh,
        out_type=jax.ShapeDtypeStruct((b_flat, D), jnp.float32),
        scratch_types=[
            pltpu.VMEM((2, K, SUB), jnp.int32),
            pltpu.VMEM((CHUNK, D), jnp.float32),
            pltpu.VMEM((CHUNK, D), jnp.float32),
            pltpu.SemaphoreType.DMA,
            pltpu.SemaphoreType.DMA,
            pltpu.SemaphoreType.DMA,
            pltpu.SemaphoreType.DMA,
            pltpu.SemaphoreType.DMA,
            pltpu.SemaphoreType.DMA,
        ],
        compiler_params=pltpu.CompilerParams(use_tc_tiling_on_sc=False),
    )
    def k(table_hbm, idx_hbm, out_hbm, idx_v, rows0, rows1,
          isem0, isem1, gsem0, gsem1, wsem0, wsem1):
        wid = lax.axis_index("s") * NC + lax.axis_index("c")
        row_base = wid * rows_per_w
        rows_v = (rows0, rows1)
        isems = (isem0, isem1)
        gsems = (gsem0, gsem1)
        wsems = (wsem0, wsem1)

        def idx_src(i):
            return idx_hbm.at[pl.ds(row_base + i * K, K), :]

        def out_dst(i):
            return out_hbm.at[pl.ds((row_base + i * K) * SUB, CHUNK), :]

        def fire_gathers(b):
            for j in range(K):
                pltpu.async_copy(
                    table_hbm.at[idx_v.at[b, j]],
                    rows_v[b].at[pl.ds(j * SUB, SUB), :],
                    gsems[b],
                )

        def drain_gathers(b):
            for j in range(K):
                pltpu.make_async_copy(
                    table_hbm.at[idx_v.at[b, j]],
                    rows_v[b].at[pl.ds(j * SUB, SUB), :],
                    gsems[b],
                ).wait()

        pltpu.async_copy(idx_src(0), idx_v.at[0], isem0)
        pltpu.async_copy(idx_src(1), idx_v.at[1], isem1)

        pltpu.make_async_copy(idx_src(0), idx_v.at[0], isem0).wait()
        fire_gathers(0)
        drain_gathers(0)
        pltpu.async_copy(idx_src(2), idx_v.at[0], isem0)
        pltpu.async_copy(rows_v[0], out_dst(0), wsems[0])

        def pair_body(p, carry):
            for b, off in ((1, 1), (0, 2)):
                i = p * 2 + off
                pltpu.make_async_copy(idx_src(i), idx_v.at[b], isems[b]).wait()
                if b == 0:
                    pltpu.make_async_copy(
                        rows_v[b], out_dst(i - 2), wsems[b]).wait()
                else:
                    @pl.when(p >= 1)
                    def _():
                        pltpu.make_async_copy(
                            rows_v[b], out_dst(i - 2), wsems[b]).wait()
                fire_gathers(b)
                drain_gathers(b)
                @pl.when(p < n_pairs - 1)
                def _():
                    pltpu.async_copy(idx_src(i + 2), idx_v.at[b], isems[b])
                pltpu.async_copy(rows_v[b], out_dst(i), wsems[b])
            return carry

        lax.fori_loop(0, n_pairs, pair_body, 0)

        pltpu.make_async_copy(rows_v[1], out_dst(n_chunks - 2), wsems[1]).wait()
        pltpu.make_async_copy(rows_v[0], out_dst(n_chunks - 1), wsems[0]).wait()

    return k(table, idx2d)


def kernel(x, table):
    b, s = x.shape
    v, d = table.shape
    n_blk = v // SUB
    tail = table[n_blk * SUB:].reshape(-1)
    table_lin = _sc_detile(table.T, tail).reshape(v, d)
    idx2d = x.T.astype(jnp.int32).reshape(-1, SUB)
    out = _sc_gather(table_lin, idx2d)
    return jnp.swapaxes(out.reshape(s, b, d), 0, 1)


# speedup vs baseline: 1.7019x; 1.7019x over previous
import functools

import jax
import jax.numpy as jnp
from jax import lax
from jax.experimental import pallas as pl
from jax.experimental.pallas import tpu as pltpu
from jax.experimental.pallas import tpu_sc as plsc

D = 32
SUB = 128
K = 4
CHUNK = K * SUB
NC = 2
NS = 16
NW = NC * NS
B = 16384
S = 50
PLANE = D * B
TROW = 8 * B


def _sc_gather_tiled(table, idx):
    n_flat = idx.shape[0]
    per_w = n_flat // NW
    n_chunks = per_w // CHUNK
    n_pairs = n_chunks // 2

    mesh = plsc.VectorSubcoreMesh(core_axis_name="c", subcore_axis_name="s")

    @functools.partial(
        pl.kernel,
        mesh=mesh,
        out_type=jax.ShapeDtypeStruct((S * PLANE,), jnp.float32),
        scratch_types=[
            pltpu.VMEM((2, CHUNK), jnp.int32),
            pltpu.VMEM((CHUNK, D), jnp.float32),
            pltpu.VMEM((CHUNK, D), jnp.float32),
            pltpu.VMEM((CHUNK * D,), jnp.float32),
            pltpu.VMEM((CHUNK * D,), jnp.float32),
            pltpu.SemaphoreType.DMA,
            pltpu.SemaphoreType.DMA,
            pltpu.SemaphoreType.DMA,
            pltpu.SemaphoreType.DMA,
            pltpu.SemaphoreType.DMA,
            pltpu.SemaphoreType.DMA,
        ],
        compiler_params=pltpu.CompilerParams(
            use_tc_tiling_on_sc=False, needs_layout_passes=False
        ),
    )
    def k(table_hbm, idx_hbm, out_hbm, idx_v, rowsa, rowsb, tilesa, tilesb,
          isem0, isem1, gsem0, gsem1, wsem0, wsem1):
        wid = lax.axis_index("s") * NC + lax.axis_index("c")
        c_base = wid * n_chunks
        rows_v = (rowsa, rowsb)
        tiles_v = (tilesa, tilesb)
        isems = (isem0, isem1)
        gsems = (gsem0, gsem1)
        wsems = (wsem0, wsem1)
        i16 = lax.iota(jnp.int32, 16)
        pats = [
            ((f0 + i16) // 8) * (K * SUB * 8) + ((f0 + i16) % 8) * SUB
            for f0 in (0, 16)
        ]

        def idx_src(c):
            return idx_hbm.at[pl.ds(c * CHUNK, CHUNK)]

        def fire_idx(c, b):
            pltpu.async_copy(idx_src(c), idx_v.at[b], isems[b])

        def wait_idx(c, b):
            pltpu.make_async_copy(idx_src(c), idx_v.at[b], isems[b]).wait()

        def fire_gathers(c, b):
            for j in range(K):
                pltpu.async_copy(
                    table_hbm.at[idx_v.at[b, pl.ds(j * SUB, SUB)]],
                    rows_v[b].at[pl.ds(j * SUB, SUB), :],
                    gsems[b],
                )

        def wait_gathers(c, b):
            for j in range(K):
                pltpu.make_async_copy(
                    table_hbm.at[idx_v.at[b, pl.ds(j * SUB, SUB)]],
                    rows_v[b].at[pl.ds(j * SUB, SUB), :],
                    gsems[b],
                ).wait()

        def out_runs(c):
            s = c // (B // CHUNK)
            tile_col = (c % (B // CHUNK)) * (CHUNK // SUB)
            base = s * PLANE + tile_col * (SUB * 8)
            return [
                (tr * (K * SUB * 8),
                 base + tr * TROW)
                for tr in range(D // 8)
            ]

        def fire_writes(c, b):
            for src_off, dst_off in out_runs(c):
                pltpu.async_copy(
                    tiles_v[b].at[pl.ds(src_off, K * SUB * 8)],
                    out_hbm.at[pl.ds(dst_off, K * SUB * 8)],
                    wsems[b],
                )

        def wait_writes(c, b):
            for src_off, dst_off in out_runs(c):
                pltpu.make_async_copy(
                    tiles_v[b].at[pl.ds(src_off, K * SUB * 8)],
                    out_hbm.at[pl.ds(dst_off, K * SUB * 8)],
                    wsems[b],
                ).wait()

        def transpose(b):
            def rows4(g, carry):
                for u in range(4):
                    r = g * 4 + u
                    base = (r // SUB) * (SUB * 8) + (r % SUB) + i16 * 0
                    for f0 in (0, 16):
                        vals = rows_v[b][r, pl.ds(f0, 16)]
                        plsc.store_scatter(
                            tiles_v[b], [pats[f0 // 16] + base], vals
                        )
                return carry

            lax.fori_loop(0, CHUNK // 4, rows4, 0)

        fire_idx(c_base, 0)
        fire_idx(c_base + 1, 1)
        wait_idx(c_base, 0)
        fire_gathers(c_base, 0)

        def pair_body(p, carry):
            for b in range(2):
                j = p * 2 + b
                c = c_base + j
                @pl.when(j < n_chunks - 1)
                def _():
                    wait_idx(c + 1, 1 - b)
                    fire_gathers(c + 1, 1 - b)

                wait_gathers(c, b)
                @pl.when(j < n_chunks - 2)
                def _():
                    fire_idx(c + 2, b)
                @pl.when(j >= 2)
                def _():
                    wait_writes(c - 2, b)

                transpose(b)
                fire_writes(c, b)
            return carry

        lax.fori_loop(0, n_pairs, pair_body, 0)
        wait_writes(c_base + n_chunks - 2, 0)
        wait_writes(c_base + n_chunks - 1, 1)

    return k(table, idx)


def kernel(x, table):
    b, s = x.shape
    v, d = table.shape
    idx = x.T.astype(jnp.int32).reshape(-1)
    flat = _sc_gather_tiled(table, idx)
    out = flat.reshape(s, d // 8, b // SUB, 8, SUB)
    out = out.transpose(2, 4, 0, 1, 3)
    return out.reshape(b, s, d)
